# baseline (speedup 1.0000x reference)
import jax
import jax.numpy as jnp
from jax.experimental import pallas as pl

_B, _N, _F, _C = 8, 2048, 512, 32


def _select_kernel(x1_ref, x2_ref, wt_ref, d_ref, d1_ref):
    wt = wt_ref[...]
    iota_n = jax.lax.broadcasted_iota(jnp.int32, (_N, _C), 0)
    for x_ref, o_ref in ((x1_ref, d_ref), (x2_ref, d1_ref)):
        x = x_ref[0]
        scores = jnp.dot(x, wt, preferred_element_type=jnp.float32)
        maxv = jnp.max(scores, axis=0, keepdims=True)
        idx = jnp.min(
            jnp.where(scores == maxv, iota_n, _N), axis=0, keepdims=True
        )
        onehot = (iota_n == idx).astype(jnp.float32)
        o_ref[0] = jax.lax.dot_general(
            onehot, x, (((0,), (0,)), ((), ())),
            preferred_element_type=jnp.float32,
            precision=jax.lax.Precision.HIGHEST,
        )


def kernel(x1, x2, W, b):
    del b
    wt = W.T
    d, d1 = pl.pallas_call(
        _select_kernel,
        grid=(_B,),
        in_specs=[
            pl.BlockSpec((1, _N, _F), lambda i: (i, 0, 0)),
            pl.BlockSpec((1, _N, _F), lambda i: (i, 0, 0)),
            pl.BlockSpec((_F, _C), lambda i: (0, 0)),
        ],
        out_specs=[
            pl.BlockSpec((1, _C, _F), lambda i: (i, 0, 0)),
            pl.BlockSpec((1, _C, _F), lambda i: (i, 0, 0)),
        ],
        out_shape=[
            jax.ShapeDtypeStruct((_B, _C, _F), jnp.float32),
            jax.ShapeDtypeStruct((_B, _C, _F), jnp.float32),
        ],
    )(x1, x2, wt)
    return (d, d1)
